# baseline (speedup 1.0000x reference)
import functools

import jax
import jax.numpy as jnp
from jax import lax
from jax.experimental import pallas as pl
from jax.experimental.pallas import tpu as pltpu
from jax.experimental.pallas import tpu_sc as plsc

NEG_SLOPE = 0.2
NEG_BIG = -1e30

B, T, N, C = 4, 12, 325, 64
H, Co = 2, 64
E = 2600
BT = B * T
EP = 2608
NG = EP // 16
NCo = N * Co
NU = BT * H
NW = 32
RPW = NU // NW


def _lin_body(x_ref, Wl0_ref, Wl1_ref, bl0_ref, bl1_ref,
              Wr0_ref, Wr1_ref, br0_ref, br1_ref, xl_ref, xr_ref):
    xsT = x_ref[0].T
    xl_ref[0, 0] = jnp.dot(Wl0_ref[...], xsT, preferred_element_type=jnp.float32) + bl0_ref[...]
    xl_ref[1, 0] = jnp.dot(Wl1_ref[...], xsT, preferred_element_type=jnp.float32) + bl1_ref[...]
    xr_ref[0, 0] = jnp.dot(Wr0_ref[...], xsT, preferred_element_type=jnp.float32) + br0_ref[...]
    xr_ref[1, 0] = jnp.dot(Wr1_ref[...], xsT, preferred_element_type=jnp.float32) + br1_ref[...]


def _post_body(agg_ref, ssum_ref, x_ref, bias0_ref, bias1_ref, WpA_ref, WpB_ref,
               bp_ref, gamma_ref, beta_ref, out_ref):
    xs = x_ref[0]
    inv0 = 1.0 / (ssum_ref[0, 0, :, :N] + 1e-16)
    inv1 = 1.0 / (ssum_ref[1, 0, :, :N] + 1e-16)
    a0 = jnp.maximum(agg_ref[0, 0] * inv0 + bias0_ref[...], 0.0)
    a1 = jnp.maximum(agg_ref[1, 0] * inv1 + bias1_ref[...], 0.0)
    y = (lax.dot_general(a0, WpA_ref[...], (((0,), (1,)), ((), ())),
                         preferred_element_type=jnp.float32)
         + lax.dot_general(a1, WpB_ref[...], (((0,), (1,)), ((), ())),
                           preferred_element_type=jnp.float32)
         + bp_ref[...] + xs)
    mu = jnp.mean(y, axis=1, keepdims=True)
    var = jnp.mean((y - mu) ** 2, axis=1, keepdims=True)
    xn = (y - mu) * lax.rsqrt(var + 1e-5)
    out_ref[0] = jnp.maximum(xn * gamma_ref[...] + beta_ref[...], 0.0)


def _sc_body(xl_hbm, xr_hbm, att_hbm, src_hbm, dst_hbm, out_hbm, ssum_hbm,
             xl_v, xr_v, agg_v, src_v, dst_v, ssum_v, att_v, attb_v, ex_v):
    wid = lax.axis_index("s") * 2 + lax.axis_index("c")

    pltpu.sync_copy(src_hbm, src_v)
    pltpu.sync_copy(dst_hbm, dst_v)
    pltpu.sync_copy(att_hbm, att_v)

    nstep = jnp.full((16,), N, jnp.int32)
    lane = lax.iota(jnp.int32, 16)

    def unit(r, _):
        u = wid + NW * r
        h = u // BT
        t = u - BT * h
        hbase = jnp.full((16,), h * Co, jnp.int32)

        pltpu.sync_copy(xl_hbm.at[h, t], xl_v)
        pltpu.sync_copy(xr_hbm.at[h, t], xr_v)

        for f in range(Co):
            attb_v[pl.ds(f * 16, 16)] = plsc.load_gather(att_v, [hbase + f])

        @plsc.parallel_loop(0, NCo // 16, unroll=4)
        def pz(i):
            agg_v[pl.ds(i * 16, 16)] = jnp.zeros((16,), jnp.float32)

        @plsc.parallel_loop(0, ssum_v.shape[0] // 16, unroll=4)
        def pzs(i):
            ssum_v[pl.ds(i * 16, 16)] = jnp.zeros((16,), jnp.float32)

        @plsc.parallel_loop(0, NG, unroll=8)
        def p1(g):
            sv = src_v[pl.ds(g * 16, 16)]
            dv = dst_v[pl.ds(g * 16, 16)]
            acc = jnp.zeros((16,), jnp.float32)
            sb, db = sv, dv
            for f in range(Co):
                xj = plsc.load_gather(xl_v, [sb])
                xi = plsc.load_gather(xr_v, [db])
                z = xi + xj
                z = jnp.maximum(z, NEG_SLOPE * z)
                acc = acc + z * attb_v[pl.ds(f * 16, 16)]
                sb = sb + nstep
                db = db + nstep
            ex = jnp.exp(acc)
            ex = jnp.where(g * 16 + lane < E, ex, 0.0)
            ex_v[pl.ds(g * 16, 16)] = ex
            plsc.addupdate_scatter(ssum_v, [dv], ex)

        @plsc.parallel_loop(0, NG, unroll=8)
        def p2(g):
            sv = src_v[pl.ds(g * 16, 16)]
            dv = dst_v[pl.ds(g * 16, 16)]
            ex = ex_v[pl.ds(g * 16, 16)]
            sb, db = sv, dv
            for f in range(Co):
                xj = plsc.load_gather(xl_v, [sb])
                plsc.addupdate_scatter(agg_v, [db], xj * ex)
                sb = sb + nstep
                db = db + nstep

        pltpu.sync_copy(agg_v, out_hbm.at[h, t])
        pltpu.sync_copy(ssum_v, ssum_hbm.at[h, t])
        return _
    lax.fori_loop(0, RPW, unit, None)


@jax.jit
def kernel(x, adj, Wl, bl, Wr, br, att, bias, Wp, bp, gamma, beta):
    x3 = x.reshape(BT, N, C)

    full = lambda *shape: pl.BlockSpec(shape, lambda i: (0,) * len(shape))
    xl4, xr4 = pl.pallas_call(
        _lin_body,
        grid=(BT,),
        in_specs=[
            pl.BlockSpec((1, N, C), lambda i: (i, 0, 0)),
            full(Co, C), full(Co, C), full(Co, 1), full(Co, 1),
            full(Co, C), full(Co, C), full(Co, 1), full(Co, 1),
        ],
        out_specs=[pl.BlockSpec((H, 1, Co, N), lambda i: (0, i, 0, 0)),
                   pl.BlockSpec((H, 1, Co, N), lambda i: (0, i, 0, 0))],
        out_shape=[jax.ShapeDtypeStruct((H, BT, Co, N), jnp.float32),
                   jax.ShapeDtypeStruct((H, BT, Co, N), jnp.float32)],
        compiler_params=pltpu.CompilerParams(
            dimension_semantics=("arbitrary",)),
    )(x3,
      Wl[:Co], Wl[Co:], bl[:Co].reshape(Co, 1), bl[Co:].reshape(Co, 1),
      Wr[:Co], Wr[Co:], br[:Co].reshape(Co, 1), br[Co:].reshape(Co, 1))

    src_p = jnp.pad(adj[0], (0, EP - E)).astype(jnp.int32)
    dst_p = jnp.pad(adj[1], (0, EP - E)).astype(jnp.int32)

    sc = functools.partial(
        pl.kernel,
        mesh=plsc.VectorSubcoreMesh(core_axis_name="c", subcore_axis_name="s"),
        compiler_params=pltpu.CompilerParams(needs_layout_passes=False),
        out_type=[jax.ShapeDtypeStruct((H, BT, NCo), jnp.float32),
                  jax.ShapeDtypeStruct((H, BT, 336), jnp.float32)],
        scratch_types=[
            pltpu.VMEM((NCo,), jnp.float32),
            pltpu.VMEM((NCo,), jnp.float32),
            pltpu.VMEM((NCo,), jnp.float32),
            pltpu.VMEM((EP,), jnp.int32),
            pltpu.VMEM((EP,), jnp.int32),
            pltpu.VMEM((336,), jnp.float32),
            pltpu.VMEM((H * Co,), jnp.float32),
            pltpu.VMEM((Co * 16,), jnp.float32),
            pltpu.VMEM((EP,), jnp.float32),
        ],
    )(_sc_body)
    agg, ssum = sc(xl4.reshape(H, BT, NCo), xr4.reshape(H, BT, NCo),
                   att.reshape(H * Co), src_p, dst_p)

    out = pl.pallas_call(
        _post_body,
        grid=(BT,),
        in_specs=[
            pl.BlockSpec((H, 1, Co, N), lambda i: (0, i, 0, 0)),
            pl.BlockSpec((H, 1, 1, 336), lambda i: (0, i, 0, 0)),
            pl.BlockSpec((1, N, C), lambda i: (i, 0, 0)),
            full(Co, 1), full(Co, 1),
            full(Co, C), full(Co, C),
            full(1, C), full(1, C), full(1, C),
        ],
        out_specs=pl.BlockSpec((1, N, C), lambda i: (i, 0, 0)),
        out_shape=jax.ShapeDtypeStruct((BT, N, C), jnp.float32),
        compiler_params=pltpu.CompilerParams(
            dimension_semantics=("arbitrary",)),
    )(agg.reshape(H, BT, Co, N), ssum.reshape(H, BT, 1, 336), x3,
      bias[:Co].reshape(Co, 1), bias[Co:].reshape(Co, 1),
      Wp[:, :Co], Wp[:, Co:],
      bp.reshape(1, C), gamma.reshape(1, C), beta.reshape(1, C))
    return out.reshape(B, T, N, C)


# speedup vs baseline: 1.1838x; 1.1838x over previous
import functools

import jax
import jax.numpy as jnp
from jax import lax
from jax.experimental import pallas as pl
from jax.experimental.pallas import tpu as pltpu
from jax.experimental.pallas import tpu_sc as plsc

NEG_SLOPE = 0.2
NEG_BIG = -1e30

B, T, N, C = 4, 12, 325, 64
H, Co = 2, 64
E = 2600
BT = B * T
EP = 2608
NG = EP // 16
NCo = N * Co
NU = BT * H
NW = 32
RPW = NU // NW


def _lin_body(x_ref, Wl0_ref, Wl1_ref, bl0_ref, bl1_ref,
              Wr0_ref, Wr1_ref, br0_ref, br1_ref, xl_ref, xr_ref):
    xsT = x_ref[0].T
    xl_ref[0, 0] = jnp.dot(Wl0_ref[...], xsT, preferred_element_type=jnp.float32) + bl0_ref[...]
    xl_ref[1, 0] = jnp.dot(Wl1_ref[...], xsT, preferred_element_type=jnp.float32) + bl1_ref[...]
    xr_ref[0, 0] = jnp.dot(Wr0_ref[...], xsT, preferred_element_type=jnp.float32) + br0_ref[...]
    xr_ref[1, 0] = jnp.dot(Wr1_ref[...], xsT, preferred_element_type=jnp.float32) + br1_ref[...]


def _post_body(agg_ref, ssum_ref, x_ref, bias0_ref, bias1_ref, WpA_ref, WpB_ref,
               bp_ref, gamma_ref, beta_ref, out_ref):
    xs = x_ref[0]
    inv0 = 1.0 / (ssum_ref[0, 0, :, :N] + 1e-16)
    inv1 = 1.0 / (ssum_ref[1, 0, :, :N] + 1e-16)
    a0 = jnp.maximum(agg_ref[0, 0] * inv0 + bias0_ref[...], 0.0)
    a1 = jnp.maximum(agg_ref[1, 0] * inv1 + bias1_ref[...], 0.0)
    y = (lax.dot_general(a0, WpA_ref[...], (((0,), (1,)), ((), ())),
                         preferred_element_type=jnp.float32)
         + lax.dot_general(a1, WpB_ref[...], (((0,), (1,)), ((), ())),
                           preferred_element_type=jnp.float32)
         + bp_ref[...] + xs)
    mu = jnp.mean(y, axis=1, keepdims=True)
    var = jnp.mean((y - mu) ** 2, axis=1, keepdims=True)
    xn = (y - mu) * lax.rsqrt(var + 1e-5)
    out_ref[0] = jnp.maximum(xn * gamma_ref[...] + beta_ref[...], 0.0)


def _sc_body(xl_hbm, xr_hbm, att_hbm, src_hbm, dst_hbm, out_hbm, ssum_hbm,
             xl_v, xr_v, agg_v, src_v, dst_v, ssum_v, att_v, attb_v, ex_v):
    wid = lax.axis_index("s") * 2 + lax.axis_index("c")

    pltpu.sync_copy(src_hbm, src_v)
    pltpu.sync_copy(dst_hbm, dst_v)
    pltpu.sync_copy(att_hbm, att_v)

    nstep = jnp.full((16,), N, jnp.int32)
    lane = lax.iota(jnp.int32, 16)

    def unit(r, _):
        u = wid + NW * r
        h = u // BT
        t = u - BT * h
        hbase = jnp.full((16,), h * Co, jnp.int32)

        pltpu.sync_copy(xl_hbm.at[h, t], xl_v)
        pltpu.sync_copy(xr_hbm.at[h, t], xr_v)

        for f in range(Co):
            attb_v[pl.ds(f * 16, 16)] = plsc.load_gather(att_v, [hbase + f])

        @plsc.parallel_loop(0, NCo // 16, unroll=4)
        def pz(i):
            agg_v[pl.ds(i * 16, 16)] = jnp.zeros((16,), jnp.float32)

        @plsc.parallel_loop(0, ssum_v.shape[0] // 16, unroll=4)
        def pzs(i):
            ssum_v[pl.ds(i * 16, 16)] = jnp.zeros((16,), jnp.float32)

        @plsc.parallel_loop(0, NG, unroll=4)
        def p1(g):
            sv = src_v[pl.ds(g * 16, 16)]
            dv = dst_v[pl.ds(g * 16, 16)]
            acc = jnp.zeros((16,), jnp.float32)
            sb, db = sv, dv
            for f in range(Co):
                xj = plsc.load_gather(xl_v, [sb])
                xi = plsc.load_gather(xr_v, [db])
                z = xi + xj
                z = jnp.maximum(z, NEG_SLOPE * z)
                acc = acc + z * attb_v[pl.ds(f * 16, 16)]
                sb = sb + nstep
                db = db + nstep
            ex = jnp.exp(acc)
            ex = jnp.where(g * 16 + lane < E, ex, 0.0)
            ex_v[pl.ds(g * 16, 16)] = ex
            plsc.addupdate_scatter(ssum_v, [dv], ex)

        @plsc.parallel_loop(0, NG, unroll=4)
        def p2(g):
            sv = src_v[pl.ds(g * 16, 16)]
            dv = dst_v[pl.ds(g * 16, 16)]
            ex = ex_v[pl.ds(g * 16, 16)]
            sb, db = sv, dv
            for f in range(Co):
                xj = plsc.load_gather(xl_v, [sb])
                plsc.addupdate_scatter(agg_v, [db], xj * ex)
                sb = sb + nstep
                db = db + nstep

        pltpu.sync_copy(agg_v, out_hbm.at[h, t])
        pltpu.sync_copy(ssum_v, ssum_hbm.at[h, t])
        return _
    lax.fori_loop(0, RPW, unit, None)


@jax.jit
def kernel(x, adj, Wl, bl, Wr, br, att, bias, Wp, bp, gamma, beta):
    x3 = x.reshape(BT, N, C)

    full = lambda *shape: pl.BlockSpec(shape, lambda i: (0,) * len(shape))
    xl4, xr4 = pl.pallas_call(
        _lin_body,
        grid=(BT,),
        in_specs=[
            pl.BlockSpec((1, N, C), lambda i: (i, 0, 0)),
            full(Co, C), full(Co, C), full(Co, 1), full(Co, 1),
            full(Co, C), full(Co, C), full(Co, 1), full(Co, 1),
        ],
        out_specs=[pl.BlockSpec((H, 1, Co, N), lambda i: (0, i, 0, 0)),
                   pl.BlockSpec((H, 1, Co, N), lambda i: (0, i, 0, 0))],
        out_shape=[jax.ShapeDtypeStruct((H, BT, Co, N), jnp.float32),
                   jax.ShapeDtypeStruct((H, BT, Co, N), jnp.float32)],
        compiler_params=pltpu.CompilerParams(
            dimension_semantics=("arbitrary",)),
    )(x3,
      Wl[:Co], Wl[Co:], bl[:Co].reshape(Co, 1), bl[Co:].reshape(Co, 1),
      Wr[:Co], Wr[Co:], br[:Co].reshape(Co, 1), br[Co:].reshape(Co, 1))

    src_p = jnp.pad(adj[0], (0, EP - E)).astype(jnp.int32)
    dst_p = jnp.pad(adj[1], (0, EP - E)).astype(jnp.int32)

    sc = functools.partial(
        pl.kernel,
        mesh=plsc.VectorSubcoreMesh(core_axis_name="c", subcore_axis_name="s"),
        compiler_params=pltpu.CompilerParams(needs_layout_passes=False),
        out_type=[jax.ShapeDtypeStruct((H, BT, NCo), jnp.float32),
                  jax.ShapeDtypeStruct((H, BT, 336), jnp.float32)],
        scratch_types=[
            pltpu.VMEM((NCo,), jnp.float32),
            pltpu.VMEM((NCo,), jnp.float32),
            pltpu.VMEM((NCo,), jnp.float32),
            pltpu.VMEM((EP,), jnp.int32),
            pltpu.VMEM((EP,), jnp.int32),
            pltpu.VMEM((336,), jnp.float32),
            pltpu.VMEM((H * Co,), jnp.float32),
            pltpu.VMEM((Co * 16,), jnp.float32),
            pltpu.VMEM((EP,), jnp.float32),
        ],
    )(_sc_body)
    agg, ssum = sc(xl4.reshape(H, BT, NCo), xr4.reshape(H, BT, NCo),
                   att.reshape(H * Co), src_p, dst_p)

    out = pl.pallas_call(
        _post_body,
        grid=(BT,),
        in_specs=[
            pl.BlockSpec((H, 1, Co, N), lambda i: (0, i, 0, 0)),
            pl.BlockSpec((H, 1, 1, 336), lambda i: (0, i, 0, 0)),
            pl.BlockSpec((1, N, C), lambda i: (i, 0, 0)),
            full(Co, 1), full(Co, 1),
            full(Co, C), full(Co, C),
            full(1, C), full(1, C), full(1, C),
        ],
        out_specs=pl.BlockSpec((1, N, C), lambda i: (i, 0, 0)),
        out_shape=jax.ShapeDtypeStruct((BT, N, C), jnp.float32),
        compiler_params=pltpu.CompilerParams(
            dimension_semantics=("arbitrary",)),
    )(agg.reshape(H, BT, Co, N), ssum.reshape(H, BT, 1, 336), x3,
      bias[:Co].reshape(Co, 1), bias[Co:].reshape(Co, 1),
      Wp[:, :Co], Wp[:, Co:],
      bp.reshape(1, C), gamma.reshape(1, C), beta.reshape(1, C))
    return out.reshape(B, T, N, C)


# speedup vs baseline: 1.5222x; 1.2858x over previous
import functools

import jax
import jax.numpy as jnp
from jax import lax
from jax.experimental import pallas as pl
from jax.experimental.pallas import tpu as pltpu
from jax.experimental.pallas import tpu_sc as plsc

NEG_SLOPE = 0.2

B, T, N, C = 4, 12, 325, 64
NP = 336
H, Co = 2, 64
E = 2600
BT = B * T
EP = 2608
NG = EP // 16
NU = BT * H
NW = 32
RPW = NU // NW
SB = 8


def _lin_body(x_ref, Wl0_ref, Wl1_ref, bl0_ref, bl1_ref,
              Wr0_ref, Wr1_ref, br0_ref, br1_ref, xl_ref, xr_ref):
    pad = jnp.zeros((Co, NP - N), jnp.float32)
    for s in range(SB):
        xsT = x_ref[s].T
        for (w0, w1, b0, b1, o) in (
                (Wl0_ref, Wl1_ref, bl0_ref, bl1_ref, xl_ref),
                (Wr0_ref, Wr1_ref, br0_ref, br1_ref, xr_ref)):
            o[0, s] = jnp.concatenate(
                [jnp.dot(w0[...], xsT, preferred_element_type=jnp.float32) + b0[...], pad], axis=1)
            o[1, s] = jnp.concatenate(
                [jnp.dot(w1[...], xsT, preferred_element_type=jnp.float32) + b1[...], pad], axis=1)


def _post_body(agg_ref, ssum_ref, x_ref, bias0_ref, bias1_ref, WpA_ref, WpB_ref,
               bp_ref, gamma_ref, beta_ref, out_ref):
    for s in range(SB):
        xs = x_ref[s]
        inv0 = 1.0 / (ssum_ref[0, s, :, :N] + 1e-16)
        inv1 = 1.0 / (ssum_ref[1, s, :, :N] + 1e-16)
        a0 = jnp.maximum(agg_ref[0, s, :, :N] * inv0 + bias0_ref[...], 0.0)
        a1 = jnp.maximum(agg_ref[1, s, :, :N] * inv1 + bias1_ref[...], 0.0)
        y = (lax.dot_general(a0, WpA_ref[...], (((0,), (1,)), ((), ())),
                             preferred_element_type=jnp.float32)
             + lax.dot_general(a1, WpB_ref[...], (((0,), (1,)), ((), ())),
                               preferred_element_type=jnp.float32)
             + bp_ref[...] + xs)
        mu = jnp.mean(y, axis=1, keepdims=True)
        var = jnp.mean((y - mu) ** 2, axis=1, keepdims=True)
        xn = (y - mu) * lax.rsqrt(var + 1e-5)
        out_ref[s] = jnp.maximum(xn * gamma_ref[...] + beta_ref[...], 0.0)


def _sc_body(xl_hbm, xr_hbm, att_hbm, src_hbm, dst_hbm, out_hbm, ssum_hbm,
             xl_v, xr_v, agg_v, src_v, dst_v, ssum_v, att_v, attb_v, ex_v):
    wid = lax.axis_index("s") * 2 + lax.axis_index("c")

    pltpu.sync_copy(src_hbm, src_v)
    pltpu.sync_copy(dst_hbm, dst_v)
    pltpu.sync_copy(att_hbm, att_v)

    lane = lax.iota(jnp.int32, 16)
    zidx = jnp.zeros((16,), jnp.int32)

    def unit(r, _):
        u = wid + NW * r
        h = u // BT
        t = u - BT * h
        hbase = jnp.full((16,), h * Co, jnp.int32)

        pltpu.sync_copy(xl_hbm.at[h, t], xl_v)
        pltpu.sync_copy(xr_hbm.at[h, t], xr_v)

        for f in range(Co):
            attb_v[pl.ds(f * 16, 16)] = plsc.load_gather(att_v, [hbase + f])

        @plsc.parallel_loop(0, NP // 16, unroll=4)
        def pz(i):
            z16 = jnp.zeros((16,), jnp.float32)
            for f in range(Co):
                agg_v[f, pl.ds(i * 16, 16)] = z16

        @plsc.parallel_loop(0, NP // 16, unroll=4)
        def pzs(i):
            ssum_v[0, pl.ds(i * 16, 16)] = jnp.zeros((16,), jnp.float32)

        @plsc.parallel_loop(0, NG, unroll=4)
        def p1(g):
            sv = src_v[pl.ds(g * 16, 16)]
            dv = dst_v[pl.ds(g * 16, 16)]
            acc = jnp.zeros((16,), jnp.float32)
            for f in range(Co):
                fi = jnp.full((16,), f, jnp.int32)
                xj = plsc.load_gather(xl_v, [fi, sv])
                xi = plsc.load_gather(xr_v, [fi, dv])
                z = xi + xj
                z = jnp.maximum(z, NEG_SLOPE * z)
                acc = acc + z * attb_v[pl.ds(f * 16, 16)]
            ex = jnp.exp(acc)
            ex = jnp.where(g * 16 + lane < E, ex, 0.0)
            ex_v[pl.ds(g * 16, 16)] = ex
            plsc.addupdate_scatter(ssum_v, [zidx, dv], ex)

        @plsc.parallel_loop(0, NG, unroll=4)
        def p2(g):
            sv = src_v[pl.ds(g * 16, 16)]
            dv = dst_v[pl.ds(g * 16, 16)]
            ex = ex_v[pl.ds(g * 16, 16)]
            for f in range(Co):
                fi = jnp.full((16,), f, jnp.int32)
                xj = plsc.load_gather(xl_v, [fi, sv])
                plsc.addupdate_scatter(agg_v, [fi, dv], xj * ex)

        pltpu.sync_copy(agg_v, out_hbm.at[h, t])
        pltpu.sync_copy(ssum_v, ssum_hbm.at[h, t])
        return _
    lax.fori_loop(0, RPW, unit, None)


@jax.jit
def kernel(x, adj, Wl, bl, Wr, br, att, bias, Wp, bp, gamma, beta):
    x3 = x.reshape(BT, N, C)

    full = lambda *shape: pl.BlockSpec(shape, lambda i: (0,) * len(shape))
    xl4, xr4 = pl.pallas_call(
        _lin_body,
        grid=(BT // SB,),
        in_specs=[
            pl.BlockSpec((SB, N, C), lambda i: (i, 0, 0)),
            full(Co, C), full(Co, C), full(Co, 1), full(Co, 1),
            full(Co, C), full(Co, C), full(Co, 1), full(Co, 1),
        ],
        out_specs=[pl.BlockSpec((H, SB, Co, NP), lambda i: (0, i, 0, 0)),
                   pl.BlockSpec((H, SB, Co, NP), lambda i: (0, i, 0, 0))],
        out_shape=[jax.ShapeDtypeStruct((H, BT, Co, NP), jnp.float32),
                   jax.ShapeDtypeStruct((H, BT, Co, NP), jnp.float32)],
        compiler_params=pltpu.CompilerParams(
            dimension_semantics=("arbitrary",)),
    )(x3,
      Wl[:Co], Wl[Co:], bl[:Co].reshape(Co, 1), bl[Co:].reshape(Co, 1),
      Wr[:Co], Wr[Co:], br[:Co].reshape(Co, 1), br[Co:].reshape(Co, 1))

    src_p = jnp.pad(adj[0], (0, EP - E)).astype(jnp.int32)
    dst_p = jnp.pad(adj[1], (0, EP - E)).astype(jnp.int32)

    sc = functools.partial(
        pl.kernel,
        mesh=plsc.VectorSubcoreMesh(core_axis_name="c", subcore_axis_name="s"),
        compiler_params=pltpu.CompilerParams(needs_layout_passes=False),
        out_type=[jax.ShapeDtypeStruct((H, BT, Co, NP), jnp.float32),
                  jax.ShapeDtypeStruct((H, BT, 1, NP), jnp.float32)],
        scratch_types=[
            pltpu.VMEM((Co, NP), jnp.float32),
            pltpu.VMEM((Co, NP), jnp.float32),
            pltpu.VMEM((Co, NP), jnp.float32),
            pltpu.VMEM((EP,), jnp.int32),
            pltpu.VMEM((EP,), jnp.int32),
            pltpu.VMEM((1, NP), jnp.float32),
            pltpu.VMEM((H * Co,), jnp.float32),
            pltpu.VMEM((Co * 16,), jnp.float32),
            pltpu.VMEM((EP,), jnp.float32),
        ],
    )(_sc_body)
    agg, ssum = sc(xl4, xr4, att.reshape(H * Co), src_p, dst_p)

    out = pl.pallas_call(
        _post_body,
        grid=(BT // SB,),
        in_specs=[
            pl.BlockSpec((H, SB, Co, NP), lambda i: (0, i, 0, 0)),
            pl.BlockSpec((H, SB, 1, NP), lambda i: (0, i, 0, 0)),
            pl.BlockSpec((SB, N, C), lambda i: (i, 0, 0)),
            full(Co, 1), full(Co, 1),
            full(Co, C), full(Co, C),
            full(1, C), full(1, C), full(1, C),
        ],
        out_specs=pl.BlockSpec((SB, N, C), lambda i: (i, 0, 0)),
        out_shape=jax.ShapeDtypeStruct((BT, N, C), jnp.float32),
        compiler_params=pltpu.CompilerParams(
            dimension_semantics=("arbitrary",)),
    )(agg, ssum, x3,
      bias[:Co].reshape(Co, 1), bias[Co:].reshape(Co, 1),
      Wp[:, :Co], Wp[:, Co:],
      bp.reshape(1, C), gamma.reshape(1, C), beta.reshape(1, C))
    return out.reshape(B, T, N, C)
